# baseline (speedup 1.0000x reference)
import jax
import jax.numpy as jnp
from jax import lax
from jax.experimental import pallas as pl
from jax.experimental.pallas import tpu as pltpu
from jax.experimental.pallas import tpu_sc as plsc

N_USERS = 25000
N_ITEMS = 25000
N = N_USERS + N_ITEMS
D = 64
HD = D // 2
E = 800000
B = 4096
N_LAYERS = 3

NC = 2
NS = 16
ROWS_PER_SUB = N // NS
E_PAD = 819200
EROWS = E_PAD // 256
EROWS_PER_SUB = EROWS // NS
N_CHUNKS = EROWS_PER_SUB
BGROUPS = B // 128 // NS
ZROWS = 125
INTER = plsc.PackFormat.INTERLEAVED


def _lightgcn_body(tbl0f, tbl0, src_st, dst2d, w2d, u_st, p_st, n_st,
                   t1, t2, t3, ps_out, ns_out, eu_out, ep_out, en_out,
                   acc, srcv, dstv, wv, rows_bf, rows_f, bidx, bmean,
                   sv, gsem, ssem, isem):
    cid = lax.axis_index("c")
    sid = lax.axis_index("s")
    zero16 = jnp.zeros((16,), jnp.float32)

    def zfill(i, _):
        rows_f[i, pl.ds(0, 16)] = zero16
        rows_f[i, pl.ds(16, 16)] = zero16
        return 0

    def zero_my_acc_range():
        lax.fori_loop(0, ZROWS, zfill, 0)
        r0 = sid * ROWS_PER_SUB
        for z in range(ROWS_PER_SUB // ZROWS):
            pltpu.sync_copy(rows_f.at[pl.ds(0, ZROWS)],
                            acc.at[pl.ds(r0 + z * ZROWS, ZROWS)])

    zero_my_acc_range()
    plsc.subcore_barrier()

    ebase = sid * EROWS_PER_SUB


    def stage_idx_async(row, c):
        hrow = ebase + c
        pltpu.async_copy(src_st.at[cid, pl.ds(hrow, 2)],
                         srcv.at[pl.ds(row, 2)], isem)
        pltpu.async_copy(dst2d.at[pl.ds(hrow, 2)],
                         dstv.at[pl.ds(row, 2)], isem)
        dw = pltpu.async_copy(w2d.at[pl.ds(hrow, 2)],
                              wv.at[pl.ds(row, 2)], isem)
        return dw

    def wait_idx(row):
        pltpu.make_async_copy(src_st.at[cid, pl.ds(0, 2)],
                              srcv.at[pl.ds(row, 2)], isem).wait()
        pltpu.make_async_copy(dst2d.at[pl.ds(0, 2)],
                              dstv.at[pl.ds(row, 2)], isem).wait()
        pltpu.make_async_copy(w2d.at[pl.ds(0, 2)],
                              wv.at[pl.ds(row, 2)], isem).wait()

    def fire_gather(tin, row, half):
        pltpu.async_copy(tin.at[srcv.at[row]],
                         rows_bf.at[pl.ds(half * 256, 256)], gsem)

    def wait_gather(tin, row, half):
        pltpu.make_async_copy(tin.at[srcv.at[row]],
                              rows_bf.at[pl.ds(half * 256, 256)], gsem).wait()

    def fire_scatter(row):
        pltpu.async_copy(rows_f.at[pl.ds(0, 256)],
                         acc.at[dstv.at[row]], ssem, add=True)

    def drain_scatter(row):
        pltpu.make_async_copy(rows_f.at[pl.ds(0, 256)],
                              acc.at[dstv.at[row]], ssem).wait()

    def multiply(row, half):
        p = half * 256

        def mul_body(g16, _):
            w16 = wv[row, pl.ds(g16 * 16, 16)]
            e0 = g16 * 16
            for jj in range(16):
                w = w16[jj]
                v = rows_bf[p + e0 + jj, pl.ds(0, 32)]
                a, b = plsc.unpack(v, format=INTER)
                rows_f[e0 + jj, pl.ds(0, 16)] = a * w
                rows_f[e0 + jj, pl.ds(16, 16)] = b * w
            return 0

        lax.fori_loop(0, 16, mul_body, 0)

    NT = N_CHUNKS // 4
    tables_in = (tbl0, t1, t2)
    tables_out = (t1, t2, t3)
    for layer in range(N_LAYERS):
        tin = tables_in[layer]
        tout = tables_out[layer]

        stage_idx_async(0, 0)
        wait_idx(0)
        fire_gather(tin, 0, 0)

        def group_body(g, _, tin=tin):
            c0 = g * 4
            @pl.when(g > 0)
            def _():
                drain_scatter(3)
            stage_idx_async(2, c0 + 2)
            fire_gather(tin, 1, 1)
            wait_gather(tin, 0, 0)
            multiply(0, 0)
            fire_scatter(0)
            wait_idx(2)
            fire_gather(tin, 2, 0)
            wait_gather(tin, 1, 1)
            drain_scatter(0)
            multiply(1, 1)
            fire_scatter(1)
            drain_scatter(1)
            @pl.when(g < NT - 1)
            def _():
                stage_idx_async(0, c0 + 4)
            fire_gather(tin, 3, 1)
            wait_gather(tin, 2, 0)
            multiply(2, 0)
            fire_scatter(2)
            @pl.when(g < NT - 1)
            def _():
                wait_idx(0)
                fire_gather(tin, 0, 0)
            wait_gather(tin, 3, 1)
            drain_scatter(2)
            multiply(3, 1)
            fire_scatter(3)
            return 0

        lax.fori_loop(0, NT, group_body, 0)
        drain_scatter(3)
        plsc.subcore_barrier()
        r0 = sid * ROWS_PER_SUB

        def pack_block(i, _):
            a = rows_f[i, pl.ds(0, 16)]
            b = rows_f[i, pl.ds(16, 16)]
            rows_bf[i, pl.ds(0, 32)] = plsc.pack(a, b, format=INTER)
            return 0

        for z in range(ROWS_PER_SUB // ZROWS):
            pltpu.sync_copy(acc.at[pl.ds(r0 + z * ZROWS, ZROWS)],
                            rows_f.at[pl.ds(0, ZROWS)])
            lax.fori_loop(0, ZROWS, pack_block, 0)
            pltpu.sync_copy(rows_bf.at[pl.ds(0, ZROWS)],
                            tout.at[pl.ds(cid * N + r0 + z * ZROWS, ZROWS)])
        zero_my_acc_range()
        plsc.subcore_barrier()

    lane = lax.iota(jnp.int32, 16)
    quarter = jnp.float32(1.0 / (N_LAYERS + 1))

    def gather_mean(idx_ref, tbl4):
        descs = []
        for t in range(4):
            descs.append(pltpu.async_copy(
                tbl4[t].at[idx_ref], rows_bf.at[pl.ds(t * 128, 128)], gsem))
        for d in descs:
            d.wait()

        def mean_body(i, _):
            a0, b0 = plsc.unpack(rows_bf[i, pl.ds(0, 32)], format=INTER)
            a1, b1 = plsc.unpack(rows_bf[i + 128, pl.ds(0, 32)], format=INTER)
            a2, b2 = plsc.unpack(rows_bf[i + 256, pl.ds(0, 32)], format=INTER)
            a3, b3 = plsc.unpack(rows_bf[i + 384, pl.ds(0, 32)], format=INTER)
            bmean[i, pl.ds(0, 16)] = ((a0 + a1) + (a2 + a3)) * quarter
            bmean[i, pl.ds(16, 16)] = ((b0 + b1) + (b2 + b3)) * quarter
            return 0

        lax.fori_loop(0, 128, mean_body, 0)

    all_tables = (tbl0, t1, t2, t3)
    for g in range(BGROUPS):
        grow = sid * BGROUPS + g
        b0 = grow * 128

        pltpu.sync_copy(u_st.at[cid, grow], bidx)
        gather_mean(bidx, all_tables)

        def copy_umean(i, _):
            for h in range(2):
                s = pl.ds(h * 16, 16)
                bmean[i + 128, s] = bmean[i, s]
            return 0

        lax.fori_loop(0, 128, copy_umean, 0)

        pltpu.async_copy(tbl0f.at[bidx], rows_f.at[pl.ds(0, 128)], gsem).wait()
        pltpu.sync_copy(rows_f.at[pl.ds(0, 128)], eu_out.at[cid, pl.ds(b0, 128)])

        def dots(g16, _):
            d0 = g16 * 16
            ridx = d0 + lane
            uidx = ridx + 128
            s = jnp.zeros((16,), jnp.float32)
            for d in range(HD):
                cd = jnp.full((16,), d, jnp.int32)
                uu = plsc.load_gather(bmean, [uidx, cd])
                vv = plsc.load_gather(bmean, [ridx, cd])
                s = s + uu * vv
            sv[pl.ds(d0, 16)] = s
            return 0

        pltpu.sync_copy(p_st.at[cid, grow], bidx)
        gather_mean(bidx, all_tables)
        pltpu.async_copy(tbl0f.at[bidx], rows_f.at[pl.ds(0, 128)], gsem).wait()
        pltpu.sync_copy(rows_f.at[pl.ds(0, 128)], ep_out.at[cid, pl.ds(b0, 128)])
        lax.fori_loop(0, 8, dots, 0)
        pltpu.sync_copy(sv, ps_out.at[cid, pl.ds(b0, 128)])

        pltpu.sync_copy(n_st.at[cid, grow], bidx)
        gather_mean(bidx, all_tables)
        pltpu.async_copy(tbl0f.at[bidx], rows_f.at[pl.ds(0, 128)], gsem).wait()
        pltpu.sync_copy(rows_f.at[pl.ds(0, 128)], en_out.at[cid, pl.ds(b0, 128)])
        lax.fori_loop(0, 8, dots, 0)
        pltpu.sync_copy(sv, ns_out.at[cid, pl.ds(b0, 128)])


@jax.jit
def _lightgcn_sc(tbl0f, tbl0, src_st, dst2d, w2d, u_st, p_st, n_st):
    mesh = plsc.VectorSubcoreMesh(core_axis_name="c", subcore_axis_name="s")
    f32 = jnp.float32
    bf16 = jnp.bfloat16
    out_type = (
        jax.ShapeDtypeStruct((NC * N, HD), bf16),
        jax.ShapeDtypeStruct((NC * N, HD), bf16),
        jax.ShapeDtypeStruct((NC * N, HD), bf16),
        jax.ShapeDtypeStruct((NC, B), f32),
        jax.ShapeDtypeStruct((NC, B), f32),
        jax.ShapeDtypeStruct((NC, B, HD), f32),
        jax.ShapeDtypeStruct((NC, B, HD), f32),
        jax.ShapeDtypeStruct((NC, B, HD), f32),
    )
    scratch = [
        pltpu.VMEM_SHARED((N, HD), f32),
        pltpu.VMEM((4, 256), jnp.int32),
        pltpu.VMEM((4, 256), jnp.int32),
        pltpu.VMEM((4, 256), f32),
        pltpu.VMEM((512, HD), bf16),
        pltpu.VMEM((256, HD), f32),
        pltpu.VMEM((128,), jnp.int32),
        pltpu.VMEM((256, HD), f32),
        pltpu.VMEM((128,), f32),
        pltpu.SemaphoreType.DMA,
        pltpu.SemaphoreType.DMA,
        pltpu.SemaphoreType.DMA,
    ]
    kern = pl.kernel(
        _lightgcn_body,
        out_type=out_type,
        mesh=mesh,
        compiler_params=pltpu.CompilerParams(
            needs_layout_passes=False, use_tc_tiling_on_sc=False),
        scratch_types=scratch,
    )
    return kern(tbl0f, tbl0, src_st, dst2d, w2d, u_st, p_st, n_st)


def kernel(user_emb, item_emb, edge_index, edge_weight, users, pos_items, neg_items):
    all_emb = jnp.concatenate([user_emb, item_emb], axis=0)
    halves = all_emb.reshape(N, NC, HD).transpose(1, 0, 2)
    tbl0f = halves.reshape(NC * N, HD)
    tbl0 = tbl0f.astype(jnp.bfloat16)

    src = edge_index[0]
    dst = edge_index[1]
    pad = E_PAD - E
    zi = jnp.zeros((pad,), jnp.int32)
    srcp = jnp.concatenate([src, zi])
    dstp = jnp.concatenate([dst, zi])
    wp = jnp.concatenate([edge_weight, jnp.zeros((pad,), jnp.float32)])
    src_st = jnp.stack([srcp, srcp + N]).reshape(NC, EROWS, 256)
    dst2d = dstp.reshape(EROWS, 256)
    w2d = wp.reshape(EROWS, 256)

    u_st = jnp.stack([users, users + N]).reshape(NC, B // 128, 128)
    p_nodes = pos_items + N_USERS
    p_st = jnp.stack([p_nodes, p_nodes + N]).reshape(NC, B // 128, 128)
    n_nodes = neg_items + N_USERS
    n_st = jnp.stack([n_nodes, n_nodes + N]).reshape(NC, B // 128, 128)

    (t1, t2, t3, ps_part, ns_part, eu, ep, en) = _lightgcn_sc(
        tbl0f, tbl0, src_st, dst2d, w2d, u_st, p_st, n_st)

    pos_scores = ps_part[0] + ps_part[1]
    neg_scores = ns_part[0] + ns_part[1]
    u_emb_0 = eu.transpose(1, 0, 2).reshape(B, D)
    pos_emb_0 = ep.transpose(1, 0, 2).reshape(B, D)
    neg_emb_0 = en.transpose(1, 0, 2).reshape(B, D)
    return (pos_scores, neg_scores, u_emb_0, pos_emb_0, neg_emb_0)


# speedup vs baseline: 1.0473x; 1.0473x over previous
import jax
import jax.numpy as jnp
from jax import lax
from jax.experimental import pallas as pl
from jax.experimental.pallas import tpu as pltpu
from jax.experimental.pallas import tpu_sc as plsc

N_USERS = 25000
N_ITEMS = 25000
N = N_USERS + N_ITEMS
D = 64
HD = D // 2
E = 800000
B = 4096
N_LAYERS = 3

NC = 2
NS = 16
ROWS_PER_SUB = N // NS
E_PAD = 819200
EROWS = E_PAD // 256
EROWS_PER_SUB = EROWS // NS
N_CHUNKS = EROWS_PER_SUB
BGROUPS = B // 128 // NS
ZROWS = 125
INTER = plsc.PackFormat.INTERLEAVED


def _lightgcn_body(tbl0f, tbl0, src_st, dst2d, w2d, u_st, p_st, n_st,
                   t1, t2, t3, ps_out, ns_out, eu_out, ep_out, en_out,
                   acc, srcv, dstv, wv, rows_bf, rows_f, bidx, bmean,
                   sv, gsem, ssem, isem):
    cid = lax.axis_index("c")
    sid = lax.axis_index("s")
    zero16 = jnp.zeros((16,), jnp.float32)

    def zfill(i, _):
        rows_f[i, pl.ds(0, 16)] = zero16
        rows_f[i, pl.ds(16, 16)] = zero16
        return 0

    def zero_my_acc_range():
        lax.fori_loop(0, ZROWS, zfill, 0)
        r0 = sid * ROWS_PER_SUB
        for z in range(ROWS_PER_SUB // ZROWS):
            pltpu.sync_copy(rows_f.at[pl.ds(0, ZROWS)],
                            acc.at[pl.ds(r0 + z * ZROWS, ZROWS)])

    zero_my_acc_range()
    plsc.subcore_barrier()

    ebase = sid * EROWS_PER_SUB


    def stage_idx_async(row, c):
        hrow = ebase + c
        pltpu.async_copy(src_st.at[cid, pl.ds(hrow, 2)],
                         srcv.at[pl.ds(row, 2)], isem)
        pltpu.async_copy(dst2d.at[pl.ds(hrow, 2)],
                         dstv.at[pl.ds(row, 2)], isem)
        dw = pltpu.async_copy(w2d.at[pl.ds(hrow, 2)],
                              wv.at[pl.ds(row, 2)], isem)
        return dw

    def wait_idx(row):
        pltpu.make_async_copy(src_st.at[cid, pl.ds(0, 2)],
                              srcv.at[pl.ds(row, 2)], isem).wait()
        pltpu.make_async_copy(dst2d.at[pl.ds(0, 2)],
                              dstv.at[pl.ds(row, 2)], isem).wait()
        pltpu.make_async_copy(w2d.at[pl.ds(0, 2)],
                              wv.at[pl.ds(row, 2)], isem).wait()

    def fire_gather(tin, row, half):
        pltpu.async_copy(tin.at[srcv.at[row]],
                         rows_bf.at[pl.ds(half * 256, 256)], gsem)

    def wait_gather(tin, row, half):
        pltpu.make_async_copy(tin.at[srcv.at[row]],
                              rows_bf.at[pl.ds(half * 256, 256)], gsem).wait()

    def fire_scatter(row):
        pltpu.async_copy(rows_f.at[pl.ds(0, 256)],
                         acc.at[dstv.at[row]], ssem, add=True)

    def drain_scatter(row):
        pltpu.make_async_copy(rows_f.at[pl.ds(0, 256)],
                              acc.at[dstv.at[row]], ssem).wait()

    def multiply(row, half):
        p = half * 256

        def mul_body(g16, _):
            w16 = wv[row, pl.ds(g16 * 16, 16)]
            e0 = g16 * 16
            for jj in range(16):
                w = w16[jj]
                v = rows_bf[p + e0 + jj, pl.ds(0, 32)]
                a, b = plsc.unpack(v, format=INTER)
                rows_f[e0 + jj, pl.ds(0, 16)] = a * w
                rows_f[e0 + jj, pl.ds(16, 16)] = b * w
            return 0

        lax.fori_loop(0, 16, mul_body, 0)

    NT = N_CHUNKS // 4
    tables_in = (tbl0, t1, t2)
    tables_out = (t1, t2, t3)
    for layer in range(N_LAYERS):
        tin = tables_in[layer]
        tout = tables_out[layer]

        stage_idx_async(0, 0)
        wait_idx(0)
        fire_gather(tin, 0, 0)

        def group_body(g, _, tin=tin):
            c0 = g * 4
            @pl.when(g > 0)
            def _():
                drain_scatter(3)
            stage_idx_async(2, c0 + 2)
            wait_gather(tin, 0, 0)
            fire_gather(tin, 1, 1)
            multiply(0, 0)
            fire_scatter(0)
            wait_gather(tin, 1, 1)
            wait_idx(2)
            fire_gather(tin, 2, 0)
            drain_scatter(0)
            multiply(1, 1)
            fire_scatter(1)
            drain_scatter(1)
            @pl.when(g < NT - 1)
            def _():
                stage_idx_async(0, c0 + 4)
            wait_gather(tin, 2, 0)
            fire_gather(tin, 3, 1)
            multiply(2, 0)
            fire_scatter(2)
            wait_gather(tin, 3, 1)

            @pl.when(g < NT - 1)
            def _():
                wait_idx(0)
                fire_gather(tin, 0, 0)
            drain_scatter(2)
            multiply(3, 1)
            fire_scatter(3)
            return 0

        lax.fori_loop(0, NT, group_body, 0)
        drain_scatter(3)
        plsc.subcore_barrier()
        r0 = sid * ROWS_PER_SUB

        def pack_block(i, _):
            a = rows_f[i, pl.ds(0, 16)]
            b = rows_f[i, pl.ds(16, 16)]
            rows_bf[i, pl.ds(0, 32)] = plsc.pack(a, b, format=INTER)
            return 0

        for z in range(ROWS_PER_SUB // ZROWS):
            pltpu.sync_copy(acc.at[pl.ds(r0 + z * ZROWS, ZROWS)],
                            rows_f.at[pl.ds(0, ZROWS)])
            lax.fori_loop(0, ZROWS, pack_block, 0)
            pltpu.sync_copy(rows_bf.at[pl.ds(0, ZROWS)],
                            tout.at[pl.ds(cid * N + r0 + z * ZROWS, ZROWS)])
        zero_my_acc_range()
        plsc.subcore_barrier()

    lane = lax.iota(jnp.int32, 16)
    quarter = jnp.float32(1.0 / (N_LAYERS + 1))

    def gather_mean(idx_ref, tbl4):
        descs = []
        for t in range(4):
            descs.append(pltpu.async_copy(
                tbl4[t].at[idx_ref], rows_bf.at[pl.ds(t * 128, 128)], gsem))
        for d in descs:
            d.wait()

        def mean_body(i, _):
            a0, b0 = plsc.unpack(rows_bf[i, pl.ds(0, 32)], format=INTER)
            a1, b1 = plsc.unpack(rows_bf[i + 128, pl.ds(0, 32)], format=INTER)
            a2, b2 = plsc.unpack(rows_bf[i + 256, pl.ds(0, 32)], format=INTER)
            a3, b3 = plsc.unpack(rows_bf[i + 384, pl.ds(0, 32)], format=INTER)
            bmean[i, pl.ds(0, 16)] = ((a0 + a1) + (a2 + a3)) * quarter
            bmean[i, pl.ds(16, 16)] = ((b0 + b1) + (b2 + b3)) * quarter
            return 0

        lax.fori_loop(0, 128, mean_body, 0)

    all_tables = (tbl0, t1, t2, t3)
    for g in range(BGROUPS):
        grow = sid * BGROUPS + g
        b0 = grow * 128

        pltpu.sync_copy(u_st.at[cid, grow], bidx)
        gather_mean(bidx, all_tables)

        def copy_umean(i, _):
            for h in range(2):
                s = pl.ds(h * 16, 16)
                bmean[i + 128, s] = bmean[i, s]
            return 0

        lax.fori_loop(0, 128, copy_umean, 0)

        pltpu.async_copy(tbl0f.at[bidx], rows_f.at[pl.ds(0, 128)], gsem).wait()
        pltpu.sync_copy(rows_f.at[pl.ds(0, 128)], eu_out.at[cid, pl.ds(b0, 128)])

        def dots(g16, _):
            d0 = g16 * 16
            ridx = d0 + lane
            uidx = ridx + 128
            s = jnp.zeros((16,), jnp.float32)
            for d in range(HD):
                cd = jnp.full((16,), d, jnp.int32)
                uu = plsc.load_gather(bmean, [uidx, cd])
                vv = plsc.load_gather(bmean, [ridx, cd])
                s = s + uu * vv
            sv[pl.ds(d0, 16)] = s
            return 0

        pltpu.sync_copy(p_st.at[cid, grow], bidx)
        gather_mean(bidx, all_tables)
        pltpu.async_copy(tbl0f.at[bidx], rows_f.at[pl.ds(0, 128)], gsem).wait()
        pltpu.sync_copy(rows_f.at[pl.ds(0, 128)], ep_out.at[cid, pl.ds(b0, 128)])
        lax.fori_loop(0, 8, dots, 0)
        pltpu.sync_copy(sv, ps_out.at[cid, pl.ds(b0, 128)])

        pltpu.sync_copy(n_st.at[cid, grow], bidx)
        gather_mean(bidx, all_tables)
        pltpu.async_copy(tbl0f.at[bidx], rows_f.at[pl.ds(0, 128)], gsem).wait()
        pltpu.sync_copy(rows_f.at[pl.ds(0, 128)], en_out.at[cid, pl.ds(b0, 128)])
        lax.fori_loop(0, 8, dots, 0)
        pltpu.sync_copy(sv, ns_out.at[cid, pl.ds(b0, 128)])


@jax.jit
def _lightgcn_sc(tbl0f, tbl0, src_st, dst2d, w2d, u_st, p_st, n_st):
    mesh = plsc.VectorSubcoreMesh(core_axis_name="c", subcore_axis_name="s")
    f32 = jnp.float32
    bf16 = jnp.bfloat16
    out_type = (
        jax.ShapeDtypeStruct((NC * N, HD), bf16),
        jax.ShapeDtypeStruct((NC * N, HD), bf16),
        jax.ShapeDtypeStruct((NC * N, HD), bf16),
        jax.ShapeDtypeStruct((NC, B), f32),
        jax.ShapeDtypeStruct((NC, B), f32),
        jax.ShapeDtypeStruct((NC, B, HD), f32),
        jax.ShapeDtypeStruct((NC, B, HD), f32),
        jax.ShapeDtypeStruct((NC, B, HD), f32),
    )
    scratch = [
        pltpu.VMEM_SHARED((N, HD), f32),
        pltpu.VMEM((4, 256), jnp.int32),
        pltpu.VMEM((4, 256), jnp.int32),
        pltpu.VMEM((4, 256), f32),
        pltpu.VMEM((512, HD), bf16),
        pltpu.VMEM((256, HD), f32),
        pltpu.VMEM((128,), jnp.int32),
        pltpu.VMEM((256, HD), f32),
        pltpu.VMEM((128,), f32),
        pltpu.SemaphoreType.DMA,
        pltpu.SemaphoreType.DMA,
        pltpu.SemaphoreType.DMA,
    ]
    kern = pl.kernel(
        _lightgcn_body,
        out_type=out_type,
        mesh=mesh,
        compiler_params=pltpu.CompilerParams(
            needs_layout_passes=False, use_tc_tiling_on_sc=False),
        scratch_types=scratch,
    )
    return kern(tbl0f, tbl0, src_st, dst2d, w2d, u_st, p_st, n_st)


def kernel(user_emb, item_emb, edge_index, edge_weight, users, pos_items, neg_items):
    all_emb = jnp.concatenate([user_emb, item_emb], axis=0)
    halves = all_emb.reshape(N, NC, HD).transpose(1, 0, 2)
    tbl0f = halves.reshape(NC * N, HD)
    tbl0 = tbl0f.astype(jnp.bfloat16)

    src = edge_index[0]
    dst = edge_index[1]
    pad = E_PAD - E
    zi = jnp.zeros((pad,), jnp.int32)
    srcp = jnp.concatenate([src, zi])
    dstp = jnp.concatenate([dst, zi])
    wp = jnp.concatenate([edge_weight, jnp.zeros((pad,), jnp.float32)])
    src_st = jnp.stack([srcp, srcp + N]).reshape(NC, EROWS, 256)
    dst2d = dstp.reshape(EROWS, 256)
    w2d = wp.reshape(EROWS, 256)

    u_st = jnp.stack([users, users + N]).reshape(NC, B // 128, 128)
    p_nodes = pos_items + N_USERS
    p_st = jnp.stack([p_nodes, p_nodes + N]).reshape(NC, B // 128, 128)
    n_nodes = neg_items + N_USERS
    n_st = jnp.stack([n_nodes, n_nodes + N]).reshape(NC, B // 128, 128)

    (t1, t2, t3, ps_part, ns_part, eu, ep, en) = _lightgcn_sc(
        tbl0f, tbl0, src_st, dst2d, w2d, u_st, p_st, n_st)

    pos_scores = ps_part[0] + ps_part[1]
    neg_scores = ns_part[0] + ns_part[1]
    u_emb_0 = eu.transpose(1, 0, 2).reshape(B, D)
    pos_emb_0 = ep.transpose(1, 0, 2).reshape(B, D)
    neg_emb_0 = en.transpose(1, 0, 2).reshape(B, D)
    return (pos_scores, neg_scores, u_emb_0, pos_emb_0, neg_emb_0)
